# baseline (speedup 1.0000x reference)
import functools

import jax
import jax.numpy as jnp
from jax import lax
from jax.experimental import pallas as pl
from jax.experimental.pallas import tpu as pltpu
from jax.experimental.pallas import tpu_sc as plsc

N_NODES = 10000
N_GRAPHS = 64
D = 128
DH = 64
D_OUT = 10
BN_EPS = 1e-5

NC = 2
NS = 16
NW = NC * NS

EDGE_K = 80
CHUNKS = 256
PHASES = 2
PCHUNKS = CHUNKS // PHASES
E_PAD = NS * CHUNKS * EDGE_K
NBUF = 8
ACC_ROWS = 10112
ROWS_PER_TILE = ACC_ROWS // NS

POOL_ROWS = 640
POOL_CHUNK = 80
FSUB = DH // 16


def _pool_rows(xh_hbm, batch_hbm, pool_hbm, xv, bv, part, cid, sid):
    start = jnp.minimum(sid * POOL_ROWS, N_NODES - POOL_ROWS)
    neg = jnp.full((16,), -jnp.inf, jnp.float32)

    def initrow(g, carry):
        for f in range(FSUB):
            part[g, pl.ds(16 * f, 16)] = neg
        return carry

    lax.fori_loop(0, N_GRAPHS, initrow, 0)

    def rowgroup(q, carry):
        bvec = bv[pl.ds(q * 16, 16)]
        for k in range(16):
            g = bvec[k]
            r = q * 16 + k
            for f in range(FSUB):
                sl = pl.ds(16 * f, 16)
                part[g, sl] = jnp.maximum(part[g, sl], xv[r, sl])
        return carry

    for j in range(POOL_ROWS // POOL_CHUNK):
        off = start + j * POOL_CHUNK
        pltpu.sync_copy(xh_hbm.at[cid, pl.ds(off, POOL_CHUNK)], xv)
        pltpu.sync_copy(batch_hbm.at[pl.ds(off, POOL_CHUNK)], bv)
        lax.fori_loop(0, POOL_CHUNK // 16, rowgroup, 0)

    pltpu.sync_copy(part, pool_hbm.at[cid, sid])


def _agg_body(do_pool, xh_hbm, src_hbm, dst_hbm, zeros_hbm, batch_hbm,
              part_hbm, pool_hbm, acc, src_v, dst_v, rows, gsems, ssems,
              xv, bv, poolp):
    cid = lax.axis_index("c")
    sid = lax.axis_index("s")

    with jax.named_scope("agg_zero"):
        base = sid * ROWS_PER_TILE
        pltpu.sync_copy(zeros_hbm.at[pl.ds(base, ROWS_PER_TILE)],
                        acc.at[pl.ds(base, ROWS_PER_TILE)])
        plsc.subcore_barrier()

    with jax.named_scope("agg_ring"):
        for ph in range(PHASES):
            pltpu.sync_copy(src_hbm.at[sid, pl.ds(ph * PCHUNKS, PCHUNKS)],
                            src_v)
            pltpu.sync_copy(dst_hbm.at[sid, pl.ds(ph * PCHUNKS, PCHUNKS)],
                            dst_v)

            for b in range(NBUF):
                pltpu.async_copy(xh_hbm.at[cid].at[src_v.at[b]], rows[b],
                                 gsems[b])

            def step(c, carry):
                first = c * NBUF
                sdescs = []
                for b in range(NBUF):
                    pltpu.make_async_copy(xh_hbm.at[cid].at[src_v.at[b]],
                                          rows[b], gsems[b]).wait()
                    sdescs.append(pltpu.async_copy(
                        rows[b], acc.at[dst_v.at[first + b]], ssems[b],
                        add=True))
                for b in range(NBUF):
                    sdescs[b].wait()
                    nxt = jnp.minimum(first + b + NBUF, PCHUNKS - 1)
                    pltpu.async_copy(xh_hbm.at[cid].at[src_v.at[nxt]],
                                     rows[b], gsems[b])
                return carry

            lax.fori_loop(0, PCHUNKS // NBUF, step, 0, unroll=False)
            for b in range(NBUF):
                pltpu.make_async_copy(xh_hbm.at[cid].at[src_v.at[b]],
                                      rows[b], gsems[b]).wait()

    if do_pool:
        with jax.named_scope("agg_pool"):
            _pool_rows(xh_hbm, batch_hbm, pool_hbm, xv, bv, poolp, cid,
                       sid)

    with jax.named_scope("agg_drain"):
        plsc.subcore_barrier()
        pltpu.sync_copy(acc.at[pl.ds(base, ROWS_PER_TILE)],
                        part_hbm.at[cid, pl.ds(base, ROWS_PER_TILE)])


def _make_agg(do_pool):
    return pl.kernel(
        functools.partial(_agg_body, do_pool),
        out_type=(
            jax.ShapeDtypeStruct((NC, ACC_ROWS, DH), jnp.float32),
            jax.ShapeDtypeStruct((NC, NS, N_GRAPHS, DH), jnp.float32),
        ),
        mesh=plsc.VectorSubcoreMesh(core_axis_name="c",
                                    subcore_axis_name="s"),
        compiler_params=pltpu.CompilerParams(use_tc_tiling_on_sc=False),
        scratch_types=[
            pltpu.VMEM_SHARED((ACC_ROWS, DH), jnp.float32),
            pltpu.VMEM((PCHUNKS, EDGE_K), jnp.int32),
            pltpu.VMEM((PCHUNKS, EDGE_K), jnp.int32),
            [pltpu.VMEM((EDGE_K, DH), jnp.float32)] * NBUF,
            [pltpu.SemaphoreType.DMA] * NBUF,
            [pltpu.SemaphoreType.DMA] * NBUF,
            pltpu.VMEM((POOL_CHUNK, DH), jnp.float32),
            pltpu.VMEM((POOL_CHUNK,), jnp.int32),
            pltpu.VMEM((N_GRAPHS, DH), jnp.float32),
        ],
    )


_agg_call = _make_agg(False)
_agg_pool_call = _make_agg(True)


def _pool_body(xh_hbm, batch_hbm, pool_hbm, xv, bv, part):
    cid = lax.axis_index("c")
    sid = lax.axis_index("s")
    _pool_rows(xh_hbm, batch_hbm, pool_hbm, xv, bv, part, cid, sid)


_pool_call = pl.kernel(
    _pool_body,
    out_type=jax.ShapeDtypeStruct((NC, NS, N_GRAPHS, DH), jnp.float32),
    mesh=plsc.VectorSubcoreMesh(core_axis_name="c", subcore_axis_name="s"),
    compiler_params=pltpu.CompilerParams(use_tc_tiling_on_sc=False),
    scratch_types=[
        pltpu.VMEM((POOL_CHUNK, DH), jnp.float32),
        pltpu.VMEM((POOL_CHUNK,), jnp.int32),
        pltpu.VMEM((N_GRAPHS, DH), jnp.float32),
    ],
)


def _mlp_body(x_ref, a_ref, w1_ref, b1_ref, w2_ref, b2_ref, o_ref):
    h = (jnp.concatenate([x_ref[0], x_ref[1]], axis=1)
         + jnp.concatenate([a_ref[0], a_ref[1]], axis=1))
    h = jnp.dot(h, w1_ref[...], preferred_element_type=jnp.float32)
    h = jnp.maximum(h + b1_ref[...], 0.0)
    h = jnp.dot(h, w2_ref[...], preferred_element_type=jnp.float32)
    h = jnp.maximum(h + b2_ref[...], 0.0)
    o_ref[0] = h[:, :DH]
    o_ref[1] = h[:, DH:]


_MLP_BLK = 2000

_mlp_call = pl.pallas_call(
    _mlp_body,
    grid=(N_NODES // _MLP_BLK,),
    in_specs=[
        pl.BlockSpec((NC, _MLP_BLK, DH), lambda i: (0, i, 0)),
        pl.BlockSpec((NC, _MLP_BLK, DH), lambda i: (0, i, 0)),
        pl.BlockSpec((D, D), lambda i: (0, 0)),
        pl.BlockSpec((1, D), lambda i: (0, 0)),
        pl.BlockSpec((D, D), lambda i: (0, 0)),
        pl.BlockSpec((1, D), lambda i: (0, 0)),
    ],
    out_specs=pl.BlockSpec((NC, _MLP_BLK, DH), lambda i: (0, i, 0)),
    out_shape=jax.ShapeDtypeStruct((NC, N_NODES, DH), jnp.float32),
)


def _head_body(p1_ref, p2_ref, p3_ref, wc1_ref, wc2_ref, wc3_ref,
               bc_ref, o_ref):
    def pool(p_ref):
        m = jnp.max(p_ref[...], axis=1)
        return jnp.concatenate([m[0], m[1]], axis=1)

    o_ref[...] = (
        jnp.dot(pool(p1_ref), wc1_ref[...],
                preferred_element_type=jnp.float32)
        + jnp.dot(pool(p2_ref), wc2_ref[...],
                  preferred_element_type=jnp.float32)
        + jnp.dot(pool(p3_ref), wc3_ref[...],
                  preferred_element_type=jnp.float32)
        + bc_ref[...])


_head_call = pl.pallas_call(
    _head_body,
    out_shape=jax.ShapeDtypeStruct((N_GRAPHS, D), jnp.float32),
)


def _fold_bn(W1, b1, g, be):
    s = g / jnp.sqrt(1.0 + BN_EPS)
    return W1 * s[None, :], (b1 * s + be)[None, :]


def kernel(x, edge_index, batch,
           l1_W1, l1_b1, l1_g, l1_be, l1_W2, l1_b2,
           l2_W1, l2_b1, l2_g, l2_be, l2_W2, l2_b2,
           l3_W1, l3_b1, l3_g, l3_be, l3_W2, l3_b2,
           Wc, bc):
    src = edge_index[0].astype(jnp.int32)
    dst = edge_index[1].astype(jnp.int32)
    n_pad = E_PAD - src.shape[0]
    src3 = jnp.pad(src, (0, n_pad)).reshape(NS, CHUNKS, EDGE_K)
    dst3 = jnp.pad(dst, (0, n_pad), constant_values=N_NODES).reshape(
        NS, CHUNKS, EDGE_K)
    zeros = jnp.zeros((ACC_ROWS, DH), jnp.float32)
    batch32 = batch.astype(jnp.int32)

    xh = jnp.stack([x[:, :DH], x[:, DH:]])

    def layer(h, W1, b1, g, be, W2, b2, pool):
        W1e, b1e = _fold_bn(W1, b1, g, be)
        call = _agg_pool_call if pool else _agg_call
        part, p = call(h, src3, dst3, zeros, batch32)
        return _mlp_call(h, part, W1e, b1e, W2, b2[None, :]), p

    x1, _ = layer(xh, l1_W1, l1_b1, l1_g, l1_be, l1_W2, l1_b2, False)
    x2, p1 = layer(x1, l2_W1, l2_b1, l2_g, l2_be, l2_W2, l2_b2, True)
    x3, p2 = layer(x2, l3_W1, l3_b1, l3_g, l3_be, l3_W2, l3_b2, True)
    p3 = _pool_call(x3, batch32)

    Wcp = jnp.zeros((3 * D, D), jnp.float32).at[:, :D_OUT].set(Wc)
    bcp = jnp.zeros((1, D), jnp.float32).at[:, :D_OUT].set(bc[None, :])
    out = _head_call(p1, p2, p3, Wcp[:D], Wcp[D:2 * D], Wcp[2 * D:], bcp)
    return out[:, :D_OUT]


# speedup vs baseline: 1.5534x; 1.5534x over previous
import functools

import jax
import jax.numpy as jnp
from jax import lax
from jax.experimental import pallas as pl
from jax.experimental.pallas import tpu as pltpu
from jax.experimental.pallas import tpu_sc as plsc

N_NODES = 10000
N_GRAPHS = 64
D = 128
DH = 64
D_OUT = 10
BN_EPS = 1e-5

NC = 2
NS = 16
NW = NC * NS

EDGE_K = 96
CHUNKS = 210
E_PAD = NS * CHUNKS * EDGE_K
NBUF = 6
ACC_ROWS = 10112
ROWS_PER_TILE = ACC_ROWS // NS

POOL_ROWS = 640
POOL_CHUNK = 80
FSUB = DH // 16


def _pool_rows(xh_hbm, batch_hbm, pool_hbm, xv, bv, part, cid, sid):
    start = jnp.minimum(sid * POOL_ROWS, N_NODES - POOL_ROWS)
    neg = jnp.full((16,), -jnp.inf, jnp.float32)

    def initrow(g, carry):
        for f in range(FSUB):
            part[g, pl.ds(16 * f, 16)] = neg
        return carry

    lax.fori_loop(0, N_GRAPHS, initrow, 0)

    def rowgroup(q, carry):
        bvec = bv[pl.ds(q * 16, 16)]
        for k in range(16):
            g = bvec[k]
            r = q * 16 + k
            for f in range(FSUB):
                sl = pl.ds(16 * f, 16)
                part[g, sl] = jnp.maximum(part[g, sl], xv[r, sl])
        return carry

    for j in range(POOL_ROWS // POOL_CHUNK):
        off = start + j * POOL_CHUNK
        pltpu.sync_copy(xh_hbm.at[cid, pl.ds(off, POOL_CHUNK)], xv)
        pltpu.sync_copy(batch_hbm.at[pl.ds(off, POOL_CHUNK)], bv)
        lax.fori_loop(0, POOL_CHUNK // 16, rowgroup, 0)

    pltpu.sync_copy(part, pool_hbm.at[cid, sid])


def _agg_body(do_pool, xh_hbm, src_hbm, dst_hbm, zeros_hbm, batch_hbm,
              part_hbm, pool_hbm, acc, src_v, dst_v, rows, gsems, ssems,
              xv, bv, poolp):
    cid = lax.axis_index("c")
    sid = lax.axis_index("s")

    with jax.named_scope("agg_zero"):
        base = sid * ROWS_PER_TILE
        pltpu.sync_copy(zeros_hbm.at[pl.ds(base, ROWS_PER_TILE)],
                        acc.at[pl.ds(base, ROWS_PER_TILE)])
        plsc.subcore_barrier()

    with jax.named_scope("agg_stage_idx"):
        pltpu.sync_copy(src_hbm.at[sid], src_v)
        pltpu.sync_copy(dst_hbm.at[sid], dst_v)

    with jax.named_scope("agg_ring"):
        for b in range(NBUF):
            pltpu.async_copy(xh_hbm.at[cid].at[src_v.at[b]], rows[b],
                             gsems[b])

        def step(c, carry):
            first = c * NBUF
            sdescs = []
            for b in range(NBUF):
                pltpu.make_async_copy(xh_hbm.at[cid].at[src_v.at[b]],
                                      rows[b], gsems[b]).wait()
                sdescs.append(pltpu.async_copy(
                    rows[b], acc.at[dst_v.at[first + b]], ssems[b],
                    add=True))
            for b in range(NBUF):
                sdescs[b].wait()
                nxt = jnp.minimum(first + b + NBUF, CHUNKS - 1)
                pltpu.async_copy(xh_hbm.at[cid].at[src_v.at[nxt]], rows[b],
                                 gsems[b])
            return carry

        lax.fori_loop(0, CHUNKS // NBUF, step, 0, unroll=False)
        for b in range(NBUF):
            pltpu.make_async_copy(xh_hbm.at[cid].at[src_v.at[b]], rows[b],
                                  gsems[b]).wait()

    if do_pool:
        with jax.named_scope("agg_pool"):
            _pool_rows(xh_hbm, batch_hbm, pool_hbm, xv, bv, poolp, cid,
                       sid)

    with jax.named_scope("agg_drain"):
        plsc.subcore_barrier()
        pltpu.sync_copy(acc.at[pl.ds(base, ROWS_PER_TILE)],
                        part_hbm.at[cid, pl.ds(base, ROWS_PER_TILE)])


def _make_agg(do_pool):
    return pl.kernel(
        functools.partial(_agg_body, do_pool),
        out_type=(
            jax.ShapeDtypeStruct((NC, ACC_ROWS, DH), jnp.float32),
            jax.ShapeDtypeStruct((NC, NS, N_GRAPHS, DH), jnp.float32),
        ),
        mesh=plsc.VectorSubcoreMesh(core_axis_name="c",
                                    subcore_axis_name="s"),
        compiler_params=pltpu.CompilerParams(use_tc_tiling_on_sc=False),
        scratch_types=[
            pltpu.VMEM_SHARED((ACC_ROWS, DH), jnp.float32),
            pltpu.VMEM((CHUNKS, EDGE_K), jnp.int32),
            pltpu.VMEM((CHUNKS, EDGE_K), jnp.int32),
            [pltpu.VMEM((EDGE_K, DH), jnp.float32)] * NBUF,
            [pltpu.SemaphoreType.DMA] * NBUF,
            [pltpu.SemaphoreType.DMA] * NBUF,
            pltpu.VMEM((POOL_CHUNK, DH), jnp.float32),
            pltpu.VMEM((POOL_CHUNK,), jnp.int32),
            pltpu.VMEM((N_GRAPHS, DH), jnp.float32),
        ],
    )


_agg_call = _make_agg(False)
_agg_pool_call = _make_agg(True)


def _pool_body(xh_hbm, batch_hbm, pool_hbm, xv, bv, part):
    cid = lax.axis_index("c")
    sid = lax.axis_index("s")
    _pool_rows(xh_hbm, batch_hbm, pool_hbm, xv, bv, part, cid, sid)


_pool_call = pl.kernel(
    _pool_body,
    out_type=jax.ShapeDtypeStruct((NC, NS, N_GRAPHS, DH), jnp.float32),
    mesh=plsc.VectorSubcoreMesh(core_axis_name="c", subcore_axis_name="s"),
    compiler_params=pltpu.CompilerParams(use_tc_tiling_on_sc=False),
    scratch_types=[
        pltpu.VMEM((POOL_CHUNK, DH), jnp.float32),
        pltpu.VMEM((POOL_CHUNK,), jnp.int32),
        pltpu.VMEM((N_GRAPHS, DH), jnp.float32),
    ],
)


def _mlp_body(x_ref, a_ref, w1_ref, b1_ref, w2_ref, b2_ref, o_ref):
    h = (jnp.concatenate([x_ref[0], x_ref[1]], axis=1)
         + jnp.concatenate([a_ref[0], a_ref[1]], axis=1))
    h = jnp.dot(h, w1_ref[...], preferred_element_type=jnp.float32)
    h = jnp.maximum(h + b1_ref[...], 0.0)
    h = jnp.dot(h, w2_ref[...], preferred_element_type=jnp.float32)
    h = jnp.maximum(h + b2_ref[...], 0.0)
    o_ref[0] = h[:, :DH]
    o_ref[1] = h[:, DH:]


_MLP_BLK = 2000

_mlp_call = pl.pallas_call(
    _mlp_body,
    grid=(N_NODES // _MLP_BLK,),
    in_specs=[
        pl.BlockSpec((NC, _MLP_BLK, DH), lambda i: (0, i, 0)),
        pl.BlockSpec((NC, _MLP_BLK, DH), lambda i: (0, i, 0)),
        pl.BlockSpec((D, D), lambda i: (0, 0)),
        pl.BlockSpec((1, D), lambda i: (0, 0)),
        pl.BlockSpec((D, D), lambda i: (0, 0)),
        pl.BlockSpec((1, D), lambda i: (0, 0)),
    ],
    out_specs=pl.BlockSpec((NC, _MLP_BLK, DH), lambda i: (0, i, 0)),
    out_shape=jax.ShapeDtypeStruct((NC, N_NODES, DH), jnp.float32),
)


def _head_body(p1_ref, p2_ref, p3_ref, wc1_ref, wc2_ref, wc3_ref,
               bc_ref, o_ref):
    def pool(p_ref):
        m = jnp.max(p_ref[...], axis=1)
        return jnp.concatenate([m[0], m[1]], axis=1)

    o_ref[...] = (
        jnp.dot(pool(p1_ref), wc1_ref[...],
                preferred_element_type=jnp.float32)
        + jnp.dot(pool(p2_ref), wc2_ref[...],
                  preferred_element_type=jnp.float32)
        + jnp.dot(pool(p3_ref), wc3_ref[...],
                  preferred_element_type=jnp.float32)
        + bc_ref[...])


_head_call = pl.pallas_call(
    _head_body,
    out_shape=jax.ShapeDtypeStruct((N_GRAPHS, D), jnp.float32),
)


def _fold_bn(W1, b1, g, be):
    s = g / jnp.sqrt(1.0 + BN_EPS)
    return W1 * s[None, :], (b1 * s + be)[None, :]


def kernel(x, edge_index, batch,
           l1_W1, l1_b1, l1_g, l1_be, l1_W2, l1_b2,
           l2_W1, l2_b1, l2_g, l2_be, l2_W2, l2_b2,
           l3_W1, l3_b1, l3_g, l3_be, l3_W2, l3_b2,
           Wc, bc):
    src = edge_index[0].astype(jnp.int32)
    dst = edge_index[1].astype(jnp.int32)
    n_pad = E_PAD - src.shape[0]
    src3 = jnp.pad(src, (0, n_pad)).reshape(NS, CHUNKS, EDGE_K)
    dst3 = jnp.pad(dst, (0, n_pad), constant_values=N_NODES).reshape(
        NS, CHUNKS, EDGE_K)
    zeros = jnp.zeros((ACC_ROWS, DH), jnp.float32)
    batch32 = batch.astype(jnp.int32)

    xh = jnp.stack([x[:, :DH], x[:, DH:]])

    def layer(h, W1, b1, g, be, W2, b2, pool):
        W1e, b1e = _fold_bn(W1, b1, g, be)
        call = _agg_pool_call if pool else _agg_call
        part, p = call(h, src3, dst3, zeros, batch32)
        return _mlp_call(h, part, W1e, b1e, W2, b2[None, :]), p

    x1, _ = layer(xh, l1_W1, l1_b1, l1_g, l1_be, l1_W2, l1_b2, False)
    x2, p1 = layer(x1, l2_W1, l2_b1, l2_g, l2_be, l2_W2, l2_b2, True)
    x3, p2 = layer(x2, l3_W1, l3_b1, l3_g, l3_be, l3_W2, l3_b2, True)
    p3 = _pool_call(x3, batch32)

    Wcp = jnp.zeros((3 * D, D), jnp.float32).at[:, :D_OUT].set(Wc)
    bcp = jnp.zeros((1, D), jnp.float32).at[:, :D_OUT].set(bc[None, :])
    out = _head_call(p1, p2, p3, Wcp[:D], Wcp[D:2 * D], Wcp[2 * D:], bcp)
    return out[:, :D_OUT]
